# Initial kernel scaffold; baseline (speedup 1.0000x reference)
import jax
import jax.numpy as jnp
from jax.experimental import pallas as pl


def kernel(x, edge_index, edge_weight, g1, b1, W1, bw1, gn, bn, Wc0, bc0, Wc1, bc1, Wc2, bc2, gf, bf, Wf1, bwf1, Wf2, bwf2):
    raise NotImplementedError("write your pallas kernel here")


# speedup vs baseline: 3.6132x; 3.6132x over previous
import functools

import jax
import jax.numpy as jnp
from jax import lax
from jax.experimental import pallas as pl
from jax.experimental.pallas import tpu as pltpu
from jax.experimental.pallas import tpu_sc as plsc

N = 10000
E = 320000
D = 128
NC = 2
NS = 16
NW = NC * NS
EPW = E // NW
C = 80
NCHUNK = EPW // C
N_PAD = 10240
RPT = N_PAD // NS

_EPS = 1e-5
_PREC = lax.Precision.HIGHEST


def _sc_body(h_hbm, src_hbm, dst_hbm, w_hbm, zeros_hbm, out_hbm,
             srcb, dstb, wb, rows, acc, gsem):
    cid = lax.axis_index("c")
    sid = lax.axis_index("s")
    wid = sid * NC + cid
    ebase = wid * EPW

    pltpu.sync_copy(zeros_hbm.at[pl.ds(sid * RPT, RPT)],
                    acc.at[pl.ds(sid * RPT, RPT)])
    plsc.subcore_barrier()

    def chunk(g, carry):
        base = ebase + g * C
        pltpu.sync_copy(src_hbm.at[pl.ds(base, C)], srcb.at[0])
        pltpu.sync_copy(dst_hbm.at[pl.ds(base, C)], dstb.at[0])
        pltpu.sync_copy(w_hbm.at[pl.ds(base, C)], wb)
        pltpu.async_copy(h_hbm.at[srcb.at[0]], rows.at[0], gsem).wait()
        for grp in range(C // 16):
            wrow = wb[pl.ds(grp * 16, 16)]
            for j in range(16):
                e = grp * 16 + j
                wspl = jnp.broadcast_to(wrow[j], (16,))
                for r in range(8):
                    sl = pl.ds(r * 16, 16)
                    rows[0, e, sl] = rows[0, e, sl] * wspl
        pltpu.sync_copy(rows.at[0], acc.at[dstb.at[0]], add=True)
        return carry

    lax.fori_loop(0, NCHUNK, chunk, 0)

    plsc.subcore_barrier()
    pltpu.sync_copy(acc.at[pl.ds(sid * RPT, RPT)],
                    out_hbm.at[cid, pl.ds(sid * RPT, RPT)])


_sc_agg_cache = []


def _sc_agg(h, src, dst, w, zeros):
    if not _sc_agg_cache:
        _sc_agg_cache.append(functools.partial(
            pl.kernel,
            out_type=jax.ShapeDtypeStruct((NC, N_PAD, D), jnp.float32),
            mesh=plsc.VectorSubcoreMesh(
                core_axis_name="c", subcore_axis_name="s",
                num_cores=NC, num_subcores=NS),
            scratch_types=[
                pltpu.VMEM((1, C), jnp.int32),
                pltpu.VMEM((1, C), jnp.int32),
                pltpu.VMEM((C,), jnp.float32),
                pltpu.VMEM((1, C, D), jnp.float32),
                pltpu.VMEM_SHARED((N_PAD, D), jnp.float32),
                pltpu.SemaphoreType.DMA,
            ],
        )(_sc_body))
    return _sc_agg_cache[0](h, src, dst, w, zeros)


def _ln(x, g, b):
    m = jnp.mean(x, axis=-1, keepdims=True)
    v = jnp.mean((x - m) ** 2, axis=-1, keepdims=True)
    return (x - m) * lax.rsqrt(v + _EPS) * g + b


def _elu(x):
    return jnp.where(x > 0, x, jnp.exp(jnp.minimum(x, 0.0)) - 1.0)


def _first_body(x_ref, g1_ref, b1_ref, w1_ref, bw1_ref, gn_ref, bn_ref, o_ref):
    x = x_ref[...]
    h = _ln(x, g1_ref[...], b1_ref[...])
    h = _elu(lax.dot_general(h, w1_ref[...], (((1,), (0,)), ((), ())),
                             precision=_PREC) + bw1_ref[...])
    o_ref[...] = _ln(h, gn_ref[...], bn_ref[...])


def _layer_body(p0_ref, p1_ref, h_ref, wc_ref, bc_ref, gn_ref, bn_ref, o_ref):
    agg = p0_ref[...] + p1_ref[...]
    conv = lax.dot_general(agg, wc_ref[...], (((1,), (0,)), ((), ())),
                           precision=_PREC) + bc_ref[...]
    o_ref[...] = _ln(conv, gn_ref[...], bn_ref[...]) + h_ref[...]


def _final_body(h_ref, gf_ref, bf_ref, w1_ref, b1_ref, w2_ref, b2_ref, o_ref):
    t = _ln(h_ref[...], gf_ref[...], bf_ref[...])
    t = _elu(lax.dot_general(t, w1_ref[...], (((1,), (0,)), ((), ())),
                             precision=_PREC) + b1_ref[...])
    o_ref[...] = lax.dot_general(t, w2_ref[...], (((1,), (0,)), ((), ())),
                                 precision=_PREC) + b2_ref[...]


_BLK = 1000
_GRID = N // _BLK


def _row_spec():
    return pl.BlockSpec((_BLK, D), lambda i: (i, 0))


def _vec_spec():
    return pl.BlockSpec((1, D), lambda i: (0, 0))


def _mat_spec():
    return pl.BlockSpec((D, D), lambda i: (0, 0))


def _tc_first(x, g1, b1, W1, bw1, gn, bn):
    return pl.pallas_call(
        _first_body,
        grid=(_GRID,),
        in_specs=[_row_spec(), _vec_spec(), _vec_spec(), _mat_spec(),
                  _vec_spec(), _vec_spec(), _vec_spec()],
        out_specs=_row_spec(),
        out_shape=jax.ShapeDtypeStruct((N, D), jnp.float32),
    )(x, g1, b1, W1, bw1, gn, bn)


def _tc_layer(p0, p1, h, Wc, bc, gn, bn):
    return pl.pallas_call(
        _layer_body,
        grid=(_GRID,),
        in_specs=[_row_spec(), _row_spec(), _row_spec(), _mat_spec(),
                  _vec_spec(), _vec_spec(), _vec_spec()],
        out_specs=_row_spec(),
        out_shape=jax.ShapeDtypeStruct((N, D), jnp.float32),
    )(p0, p1, h, Wc, bc, gn, bn)


def _tc_final(h, gf, bf, Wf1, bwf1, Wf2, bwf2):
    return pl.pallas_call(
        _final_body,
        grid=(_GRID,),
        in_specs=[_row_spec(), _vec_spec(), _vec_spec(), _mat_spec(),
                  _vec_spec(), _mat_spec(), _vec_spec()],
        out_specs=_row_spec(),
        out_shape=jax.ShapeDtypeStruct((N, D), jnp.float32),
    )(h, gf, bf, Wf1, bwf1, Wf2, bwf2)


def kernel(x, edge_index, edge_weight, g1, b1, W1, bw1, gn, bn,
           Wc0, bc0, Wc1, bc1, Wc2, bc2, gf, bf, Wf1, bwf1, Wf2, bwf2):
    src = edge_index[0]
    dst = edge_index[1]
    zeros = jnp.zeros((N_PAD, D), jnp.float32)

    r = lambda v: v.reshape(1, D)
    h = _tc_first(x, r(g1), r(b1), W1, r(bw1), r(gn), r(bn))

    for Wc, bc in ((Wc0, bc0), (Wc1, bc1), (Wc2, bc2)):
        parts = _sc_agg(h, src, dst, edge_weight, zeros)
        h = _tc_layer(parts[0], parts[1], h, Wc, r(bc), r(gn), r(bn))

    return _tc_final(h, r(gf), r(bf), Wf1, r(bwf1), Wf2, r(bwf2))


# speedup vs baseline: 7.5187x; 2.0809x over previous
import functools

import jax
import jax.numpy as jnp
from jax import lax
from jax.experimental import pallas as pl
from jax.experimental.pallas import tpu as pltpu
from jax.experimental.pallas import tpu_sc as plsc

N = 10000
E = 320000
D = 128
DH = D // 2
NC = 2
NS = 16
EPT = E // NS
C = 80
NCHUNK = EPT // C
NBUF = 5
PREF = 2
NOUT = NCHUNK // NBUF
DRT = 10
RPD = N // DRT

_EPS = 1e-5
_PREC = lax.Precision.HIGHEST


def _mul_chunk(rows, wb, g, b):
    for grp in range(C // 16):
        wrow = wb[g, pl.ds(grp * 16, 16)]
        for j in range(16):
            e = grp * 16 + j
            wspl = jnp.broadcast_to(wrow[j], (16,))
            for r in range(DH // 16):
                sl = pl.ds(r * 16, 16)
                rows[b, e, sl] = rows[b, e, sl] * wspl


def _sc_body(h2_hbm, src_hbm, dst_hbm, w_hbm, zeros_hbm, out_hbm,
             srcb, wb, sidx, dstb, rows, acc, *sems):
    gsem = sems[:NBUF]
    ssem = sems[NBUF:2 * NBUF]
    dsem = sems[2 * NBUF:]
    cid = lax.axis_index("c")
    sid = lax.axis_index("s")

    @pl.when(sid < DRT)
    def _():
        pltpu.async_copy(zeros_hbm.at[pl.ds(sid * RPD, RPD)],
                         acc.at[pl.ds(sid * RPD, RPD)], ssem[0]).wait()

    pltpu.sync_copy(src_hbm.at[sid], srcb)
    pltpu.sync_copy(w_hbm.at[sid], wb)
    plsc.subcore_barrier()

    def _gather(g, slot):
        return pltpu.make_async_copy(h2_hbm.at[sidx.at[slot]],
                                     rows.at[slot], gsem[slot])

    def _scatter(g, slot):
        return pltpu.make_async_copy(rows.at[slot], acc.at[dstb.at[slot]],
                                     ssem[slot])

    def _dfetch(g, slot):
        return pltpu.make_async_copy(dst_hbm.at[sid, g], dstb.at[slot],
                                     dsem[slot])

    def _prep_gather(g, slot):
        for grp in range(C // 16):
            sl = pl.ds(grp * 16, 16)
            sidx[slot, sl] = srcb[g, sl] * 2 + cid
        _dfetch(g, slot).start()
        _gather(g, slot).start()

    for p in range(PREF):
        _prep_gather(p, p)

    def outer(t, carry):
        for b in range(NBUF):
            g = t * NBUF + b
            pb = (b + PREF) % NBUF
            pre = g + PREF

            @pl.when(pre < NCHUNK)
            def _():
                @pl.when(pre >= NBUF)
                def _():
                    _scatter(pre - NBUF, pb).wait()
                _prep_gather(pre, pb)

            _gather(g, b).wait()
            _mul_chunk(rows, wb, g, b)
            _dfetch(g, b).wait()
            _scatter(g, b).start(add=True)
        return carry

    lax.fori_loop(0, NOUT, outer, 0)

    for b in range(NBUF):
        _scatter(NCHUNK - NBUF + b, b).wait()

    plsc.subcore_barrier()

    @pl.when(sid < DRT)
    def _():
        pltpu.sync_copy(acc.at[pl.ds(sid * RPD, RPD)],
                        out_hbm.at[cid, pl.ds(sid * RPD, RPD)])


_sc_agg_cache = []


def _sc_agg(h, src, dst, w, zeros):
    if not _sc_agg_cache:
        _sc_agg_cache.append(functools.partial(
            pl.kernel,
            out_type=jax.ShapeDtypeStruct((NC, N, DH), jnp.float32),
            mesh=plsc.VectorSubcoreMesh(
                core_axis_name="c", subcore_axis_name="s",
                num_cores=NC, num_subcores=NS),
            compiler_params=pltpu.CompilerParams(use_tc_tiling_on_sc=False),
            scratch_types=[
                pltpu.VMEM((NCHUNK, C), jnp.int32),
                pltpu.VMEM((NCHUNK, C), jnp.float32),
                pltpu.VMEM((NBUF, C), jnp.int32),
                pltpu.VMEM((NBUF, C), jnp.int32),
                pltpu.VMEM((NBUF, C, DH), jnp.float32),
                pltpu.VMEM_SHARED((N, DH), jnp.float32),
            ] + [pltpu.SemaphoreType.DMA] * (3 * NBUF),
        )(_sc_body))
    return _sc_agg_cache[0](
        h.reshape(N * NC, DH), src.reshape(NS, NCHUNK, C),
        dst.reshape(NS, NCHUNK, C), w.reshape(NS, NCHUNK, C), zeros)


def _ln(x, g, b):
    m = jnp.mean(x, axis=-1, keepdims=True)
    v = jnp.mean((x - m) ** 2, axis=-1, keepdims=True)
    return (x - m) * lax.rsqrt(v + _EPS) * g + b


def _elu(x):
    return jnp.where(x > 0, x, jnp.exp(jnp.minimum(x, 0.0)) - 1.0)


def _first_body(x_ref, g1_ref, b1_ref, w1_ref, bw1_ref, gn_ref, bn_ref, o_ref):
    x = x_ref[...]
    h = _ln(x, g1_ref[...], b1_ref[...])
    h = _elu(lax.dot_general(h, w1_ref[...], (((1,), (0,)), ((), ())),
                             precision=_PREC) + bw1_ref[...])
    o_ref[...] = _ln(h, gn_ref[...], bn_ref[...])


def _layer_body(p0_ref, p1_ref, h_ref, wc_ref, bc_ref, gn_ref, bn_ref, o_ref):
    agg = jnp.concatenate([p0_ref[...], p1_ref[...]], axis=1)
    conv = lax.dot_general(agg, wc_ref[...], (((1,), (0,)), ((), ())),
                           precision=_PREC) + bc_ref[...]
    o_ref[...] = _ln(conv, gn_ref[...], bn_ref[...]) + h_ref[...]


def _final_body(h_ref, gf_ref, bf_ref, w1_ref, b1_ref, w2_ref, b2_ref, o_ref):
    t = _ln(h_ref[...], gf_ref[...], bf_ref[...])
    t = _elu(lax.dot_general(t, w1_ref[...], (((1,), (0,)), ((), ())),
                             precision=_PREC) + b1_ref[...])
    o_ref[...] = lax.dot_general(t, w2_ref[...], (((1,), (0,)), ((), ())),
                                 precision=_PREC) + b2_ref[...]


_BLK = 1000
_GRID = N // _BLK


def _row_spec():
    return pl.BlockSpec((_BLK, D), lambda i: (i, 0))


def _vec_spec():
    return pl.BlockSpec((1, D), lambda i: (0, 0))


def _mat_spec():
    return pl.BlockSpec((D, D), lambda i: (0, 0))


def _tc_first(x, g1, b1, W1, bw1, gn, bn):
    return pl.pallas_call(
        _first_body,
        grid=(_GRID,),
        in_specs=[_row_spec(), _vec_spec(), _vec_spec(), _mat_spec(),
                  _vec_spec(), _vec_spec(), _vec_spec()],
        out_specs=_row_spec(),
        out_shape=jax.ShapeDtypeStruct((N, D), jnp.float32),
    )(x, g1, b1, W1, bw1, gn, bn)


def _half_spec():
    return pl.BlockSpec((_BLK, DH), lambda i: (i, 0))


def _tc_layer(p0, p1, h, Wc, bc, gn, bn):
    return pl.pallas_call(
        _layer_body,
        grid=(_GRID,),
        in_specs=[_half_spec(), _half_spec(), _row_spec(), _mat_spec(),
                  _vec_spec(), _vec_spec(), _vec_spec()],
        out_specs=_row_spec(),
        out_shape=jax.ShapeDtypeStruct((N, D), jnp.float32),
    )(p0, p1, h, Wc, bc, gn, bn)


def _tc_final(h, gf, bf, Wf1, bwf1, Wf2, bwf2):
    return pl.pallas_call(
        _final_body,
        grid=(_GRID,),
        in_specs=[_row_spec(), _vec_spec(), _vec_spec(), _mat_spec(),
                  _vec_spec(), _mat_spec(), _vec_spec()],
        out_specs=_row_spec(),
        out_shape=jax.ShapeDtypeStruct((N, D), jnp.float32),
    )(h, gf, bf, Wf1, bwf1, Wf2, bwf2)


def kernel(x, edge_index, edge_weight, g1, b1, W1, bw1, gn, bn,
           Wc0, bc0, Wc1, bc1, Wc2, bc2, gf, bf, Wf1, bwf1, Wf2, bwf2):
    src = edge_index[0]
    dst = edge_index[1]
    zeros = jnp.zeros((N, DH), jnp.float32)

    r = lambda v: v.reshape(1, D)
    h = _tc_first(x, r(g1), r(b1), W1, r(bw1), r(gn), r(bn))

    for Wc, bc in ((Wc0, bc0), (Wc1, bc1), (Wc2, bc2)):
        parts = _sc_agg(h, src, dst, edge_weight, zeros)
        h = _tc_layer(parts[0], parts[1], h, Wc, r(bc), r(gn), r(bn))

    return _tc_final(h, r(gf), r(bf), Wf1, r(bwf1), Wf2, r(bwf2))


# speedup vs baseline: 8.1702x; 1.0867x over previous
import functools

import jax
import jax.numpy as jnp
from jax import lax
from jax.experimental import pallas as pl
from jax.experimental.pallas import tpu as pltpu
from jax.experimental.pallas import tpu_sc as plsc

N = 10000
E = 320000
D = 128
DH = D // 2
NC = 2
NS = 16
EPT = E // NS
C = 80
NCHUNK = EPT // C
NBUF = 5
PREF = 3
NOUT = NCHUNK // NBUF
DRT = 10
RPD = N // DRT

_EPS = 1e-5
_PREC = lax.Precision.DEFAULT


def _mul_chunk(rows, wb, g, b):
    for grp in range(C // 16):
        wrow = wb[g, pl.ds(grp * 16, 16)]
        for j in range(16):
            e = grp * 16 + j
            wspl = jnp.broadcast_to(wrow[j], (16,))
            for r in range(DH // 16):
                sl = pl.ds(r * 16, 16)
                rows[b, e, sl] = rows[b, e, sl] * wspl


def _sc_body(h2_hbm, src_hbm, dst_hbm, w_hbm, zeros_hbm, out_hbm,
             srcb, wb, sidx, dstb, rows, acc, *sems):
    gsem = sems[:NBUF]
    ssem = sems[NBUF:2 * NBUF]
    dsem = sems[2 * NBUF:]
    cid = lax.axis_index("c")
    sid = lax.axis_index("s")

    @pl.when(sid < DRT)
    def _():
        pltpu.make_async_copy(zeros_hbm.at[pl.ds(sid * RPD, RPD)],
                              acc.at[pl.ds(sid * RPD, RPD)], ssem[0]).start()
    pltpu.sync_copy(src_hbm.at[sid], srcb)
    pltpu.sync_copy(w_hbm.at[sid], wb)

    @pl.when(sid < DRT)
    def _():
        pltpu.make_async_copy(zeros_hbm.at[pl.ds(sid * RPD, RPD)],
                              acc.at[pl.ds(sid * RPD, RPD)], ssem[0]).wait()
    plsc.subcore_barrier()

    def _gather(g, slot):
        return pltpu.make_async_copy(h2_hbm.at[sidx.at[slot]],
                                     rows.at[slot], gsem[slot])

    def _scatter(g, slot):
        return pltpu.make_async_copy(rows.at[slot], acc.at[dstb.at[slot]],
                                     ssem[slot])

    def _dfetch(g, slot):
        return pltpu.make_async_copy(dst_hbm.at[sid, g], dstb.at[slot],
                                     dsem[slot])

    def _prep_gather(g, slot):
        for grp in range(C // 16):
            sl = pl.ds(grp * 16, 16)
            sidx[slot, sl] = srcb[g, sl] * 2 + cid
        _dfetch(g, slot).start()
        _gather(g, slot).start()

    for p in range(PREF):
        _prep_gather(p, p)

    def outer(t, carry):
        for b in range(NBUF):
            g = t * NBUF + b
            pb = (b + PREF) % NBUF
            pre = g + PREF

            @pl.when(pre < NCHUNK)
            def _():
                @pl.when(pre >= NBUF)
                def _():
                    _scatter(pre - NBUF, pb).wait()
                _prep_gather(pre, pb)

            _gather(g, b).wait()
            _mul_chunk(rows, wb, g, b)
            _dfetch(g, b).wait()
            _scatter(g, b).start(add=True)
        return carry

    lax.fori_loop(0, NOUT, outer, 0)

    for b in range(NBUF):
        _scatter(NCHUNK - NBUF + b, b).wait()

    plsc.subcore_barrier()

    @pl.when(sid < DRT)
    def _():
        pltpu.sync_copy(acc.at[pl.ds(sid * RPD, RPD)],
                        out_hbm.at[cid, pl.ds(sid * RPD, RPD)])


_sc_agg_cache = []


def _sc_agg(h, src, dst, w, zeros):
    if not _sc_agg_cache:
        _sc_agg_cache.append(functools.partial(
            pl.kernel,
            out_type=jax.ShapeDtypeStruct((NC, N, DH), jnp.float32),
            mesh=plsc.VectorSubcoreMesh(
                core_axis_name="c", subcore_axis_name="s",
                num_cores=NC, num_subcores=NS),
            compiler_params=pltpu.CompilerParams(use_tc_tiling_on_sc=False),
            scratch_types=[
                pltpu.VMEM((NCHUNK, C), jnp.int32),
                pltpu.VMEM((NCHUNK, C), jnp.float32),
                pltpu.VMEM((NBUF, C), jnp.int32),
                pltpu.VMEM((NBUF, C), jnp.int32),
                pltpu.VMEM((NBUF, C, DH), jnp.float32),
                pltpu.VMEM_SHARED((N, DH), jnp.float32),
            ] + [pltpu.SemaphoreType.DMA] * (3 * NBUF),
        )(_sc_body))
    return _sc_agg_cache[0](
        h.reshape(N * NC, DH), src.reshape(NS, NCHUNK, C),
        dst.reshape(NS, NCHUNK, C), w.reshape(NS, NCHUNK, C), zeros)


def _ln(x, g, b):
    m = jnp.mean(x, axis=-1, keepdims=True)
    v = jnp.mean((x - m) ** 2, axis=-1, keepdims=True)
    return (x - m) * lax.rsqrt(v + _EPS) * g + b


def _elu(x):
    return jnp.where(x > 0, x, jnp.exp(jnp.minimum(x, 0.0)) - 1.0)


def _first_body(x_ref, g1_ref, b1_ref, w1_ref, bw1_ref, gn_ref, bn_ref, o_ref):
    x = x_ref[...]
    h = _ln(x, g1_ref[...], b1_ref[...])
    h = _elu(lax.dot_general(h, w1_ref[...], (((1,), (0,)), ((), ())),
                             precision=_PREC) + bw1_ref[...])
    o_ref[...] = _ln(h, gn_ref[...], bn_ref[...])


def _layer_body(p0_ref, p1_ref, h_ref, wc_ref, bc_ref, gn_ref, bn_ref, o_ref):
    agg = jnp.concatenate([p0_ref[...], p1_ref[...]], axis=1)
    conv = lax.dot_general(agg, wc_ref[...], (((1,), (0,)), ((), ())),
                           precision=_PREC) + bc_ref[...]
    o_ref[...] = _ln(conv, gn_ref[...], bn_ref[...]) + h_ref[...]


def _last_body(p0_ref, p1_ref, h_ref, wc_ref, bc_ref, gn_ref, bn_ref,
               gf_ref, bf_ref, w1_ref, b1_ref, w2_ref, b2_ref, o_ref):
    agg = jnp.concatenate([p0_ref[...], p1_ref[...]], axis=1)
    conv = lax.dot_general(agg, wc_ref[...], (((1,), (0,)), ((), ())),
                           precision=_PREC) + bc_ref[...]
    h = _ln(conv, gn_ref[...], bn_ref[...]) + h_ref[...]
    t = _ln(h, gf_ref[...], bf_ref[...])
    t = _elu(lax.dot_general(t, w1_ref[...], (((1,), (0,)), ((), ())),
                             precision=_PREC) + b1_ref[...])
    o_ref[...] = lax.dot_general(t, w2_ref[...], (((1,), (0,)), ((), ())),
                                 precision=_PREC) + b2_ref[...]


def _final_body(h_ref, gf_ref, bf_ref, w1_ref, b1_ref, w2_ref, b2_ref, o_ref):
    t = _ln(h_ref[...], gf_ref[...], bf_ref[...])
    t = _elu(lax.dot_general(t, w1_ref[...], (((1,), (0,)), ((), ())),
                             precision=_PREC) + b1_ref[...])
    o_ref[...] = lax.dot_general(t, w2_ref[...], (((1,), (0,)), ((), ())),
                                 precision=_PREC) + b2_ref[...]


_BLK = 1000
_GRID = N // _BLK


def _row_spec():
    return pl.BlockSpec((_BLK, D), lambda i: (i, 0))


def _vec_spec():
    return pl.BlockSpec((1, D), lambda i: (0, 0))


def _mat_spec():
    return pl.BlockSpec((D, D), lambda i: (0, 0))


def _tc_first(x, g1, b1, W1, bw1, gn, bn):
    return pl.pallas_call(
        _first_body,
        grid=(_GRID,),
        in_specs=[_row_spec(), _vec_spec(), _vec_spec(), _mat_spec(),
                  _vec_spec(), _vec_spec(), _vec_spec()],
        out_specs=_row_spec(),
        out_shape=jax.ShapeDtypeStruct((N, D), jnp.float32),
    )(x, g1, b1, W1, bw1, gn, bn)


def _half_spec():
    return pl.BlockSpec((_BLK, DH), lambda i: (i, 0))


def _tc_layer(p0, p1, h, Wc, bc, gn, bn):
    return pl.pallas_call(
        _layer_body,
        grid=(_GRID,),
        in_specs=[_half_spec(), _half_spec(), _row_spec(), _mat_spec(),
                  _vec_spec(), _vec_spec(), _vec_spec()],
        out_specs=_row_spec(),
        out_shape=jax.ShapeDtypeStruct((N, D), jnp.float32),
    )(p0, p1, h, Wc, bc, gn, bn)


def _tc_final(h, gf, bf, Wf1, bwf1, Wf2, bwf2):
    return pl.pallas_call(
        _final_body,
        grid=(_GRID,),
        in_specs=[_row_spec(), _vec_spec(), _vec_spec(), _mat_spec(),
                  _vec_spec(), _mat_spec(), _vec_spec()],
        out_specs=_row_spec(),
        out_shape=jax.ShapeDtypeStruct((N, D), jnp.float32),
    )(h, gf, bf, Wf1, bwf1, Wf2, bwf2)


def _tc_last(p0, p1, h, Wc, bc, gn, bn, gf, bf, Wf1, bwf1, Wf2, bwf2):
    return pl.pallas_call(
        _last_body,
        grid=(_GRID,),
        in_specs=[_half_spec(), _half_spec(), _row_spec(), _mat_spec(),
                  _vec_spec(), _vec_spec(), _vec_spec(), _vec_spec(),
                  _vec_spec(), _mat_spec(), _vec_spec(), _mat_spec(),
                  _vec_spec()],
        out_specs=_row_spec(),
        out_shape=jax.ShapeDtypeStruct((N, D), jnp.float32),
    )(p0, p1, h, Wc, bc, gn, bn, gf, bf, Wf1, bwf1, Wf2, bwf2)


def kernel(x, edge_index, edge_weight, g1, b1, W1, bw1, gn, bn,
           Wc0, bc0, Wc1, bc1, Wc2, bc2, gf, bf, Wf1, bwf1, Wf2, bwf2):
    src = edge_index[0]
    dst = edge_index[1]
    zeros = jnp.zeros((N, DH), jnp.float32)

    r = lambda v: v.reshape(1, D)
    h = _tc_first(x, r(g1), r(b1), W1, r(bw1), r(gn), r(bn))

    for Wc, bc in ((Wc0, bc0), (Wc1, bc1)):
        parts = _sc_agg(h, src, dst, edge_weight, zeros)
        h = _tc_layer(parts[0], parts[1], h, Wc, r(bc), r(gn), r(bn))

    parts = _sc_agg(h, src, dst, edge_weight, zeros)
    return _tc_last(parts[0], parts[1], h, Wc2, r(bc2), r(gn), r(bn),
                    r(gf), r(bf), Wf1, r(bwf1), Wf2, r(bwf2))


# speedup vs baseline: 8.3347x; 1.0201x over previous
import functools

import jax
import jax.numpy as jnp
from jax import lax
from jax.experimental import pallas as pl
from jax.experimental.pallas import tpu as pltpu
from jax.experimental.pallas import tpu_sc as plsc

N = 10000
E = 320000
D = 128
DH = D // 2
NC = 2
NS = 16
EPT = E // NS
C = 80
NCHUNK = EPT // C
NBUF = 5
PREF = 3
NOUT = NCHUNK // NBUF
DRT = 10
RPD = N // DRT

_EPS = 1e-5
_PREC = lax.Precision.DEFAULT


def _mul_chunk(rows_i, rows_f, wb, g, b):
    mask = jnp.full((16,), -65536, jnp.int32)
    for grp in range(C // 16):
        wrow = wb[g, pl.ds(grp * 16, 16)]
        for j in range(16):
            e = grp * 16 + j
            wspl = jnp.broadcast_to(wrow[j], (16,))
            for q in range(DH // 32):
                xi = rows_i[b, e, pl.ds(q * 16, 16)]
                lo = lax.bitcast_convert_type(xi << 16, jnp.float32)
                hi = lax.bitcast_convert_type(xi & mask, jnp.float32)
                rows_f[b, e, pl.ds(q * 32, 16)] = lo * wspl
                rows_f[b, e, pl.ds(q * 32 + 16, 16)] = hi * wspl


def _sc_body(h2_hbm, src_hbm, dst_hbm, w_hbm, zeros_hbm, out_hbm,
             srcb, wb, sidx, dstb, rows_i, rows_f, acc, *sems):
    gsem = sems[:NBUF]
    ssem = sems[NBUF:2 * NBUF]
    dsem = sems[2 * NBUF:]
    cid = lax.axis_index("c")
    sid = lax.axis_index("s")

    @pl.when(sid < DRT)
    def _():
        pltpu.make_async_copy(zeros_hbm.at[pl.ds(sid * RPD, RPD)],
                              acc.at[pl.ds(sid * RPD, RPD)], ssem[0]).start()
    pltpu.sync_copy(src_hbm.at[sid], srcb)
    pltpu.sync_copy(w_hbm.at[sid], wb)

    @pl.when(sid < DRT)
    def _():
        pltpu.make_async_copy(zeros_hbm.at[pl.ds(sid * RPD, RPD)],
                              acc.at[pl.ds(sid * RPD, RPD)], ssem[0]).wait()
    plsc.subcore_barrier()

    def _gather(g, slot):
        return pltpu.make_async_copy(h2_hbm.at[sidx.at[slot]],
                                     rows_i.at[slot], gsem[slot])

    def _scatter(g, slot):
        return pltpu.make_async_copy(rows_f.at[slot], acc.at[dstb.at[slot]],
                                     ssem[slot])

    def _dfetch(g, slot):
        return pltpu.make_async_copy(dst_hbm.at[sid, g], dstb.at[slot],
                                     dsem[slot])

    def _prep_gather(g, slot):
        for grp in range(C // 16):
            sl = pl.ds(grp * 16, 16)
            sidx[slot, sl] = srcb[g, sl] * 2 + cid
        _dfetch(g, slot).start()
        _gather(g, slot).start()

    for p in range(PREF):
        _prep_gather(p, p)

    def outer(t, carry):
        for b in range(NBUF):
            g = t * NBUF + b
            pb = (b + PREF) % NBUF
            pre = g + PREF

            @pl.when(pre < NCHUNK)
            def _():
                @pl.when(pre >= NBUF)
                def _():
                    _scatter(pre - NBUF, pb).wait()
                _prep_gather(pre, pb)

            _gather(g, b).wait()
            _mul_chunk(rows_i, rows_f, wb, g, b)
            _dfetch(g, b).wait()
            _scatter(g, b).start(add=True)
        return carry

    lax.fori_loop(0, NOUT, outer, 0)

    for b in range(NBUF):
        _scatter(NCHUNK - NBUF + b, b).wait()

    plsc.subcore_barrier()

    @pl.when(sid < DRT)
    def _():
        pltpu.sync_copy(acc.at[pl.ds(sid * RPD, RPD)],
                        out_hbm.at[cid, pl.ds(sid * RPD, RPD)])


_sc_agg_cache = []


def _sc_agg(h, src, dst, w, zeros):
    if not _sc_agg_cache:
        _sc_agg_cache.append(functools.partial(
            pl.kernel,
            out_type=jax.ShapeDtypeStruct((NC, N, DH), jnp.float32),
            mesh=plsc.VectorSubcoreMesh(
                core_axis_name="c", subcore_axis_name="s",
                num_cores=NC, num_subcores=NS),
            compiler_params=pltpu.CompilerParams(use_tc_tiling_on_sc=False),
            scratch_types=[
                pltpu.VMEM((NCHUNK, C), jnp.int32),
                pltpu.VMEM((NCHUNK, C), jnp.float32),
                pltpu.VMEM((NBUF, C), jnp.int32),
                pltpu.VMEM((NBUF, C), jnp.int32),
                pltpu.VMEM((NBUF, C, DH // 2), jnp.int32),
                pltpu.VMEM((NBUF, C, DH), jnp.float32),
                pltpu.VMEM_SHARED((N, DH), jnp.float32),
            ] + [pltpu.SemaphoreType.DMA] * (3 * NBUF),
        )(_sc_body))
    return _sc_agg_cache[0](
        h.reshape(N * NC, DH // 2), src.reshape(NS, NCHUNK, C),
        dst.reshape(NS, NCHUNK, C), w.reshape(NS, NCHUNK, C), zeros)


def _pack_h(h):
    u = lax.bitcast_convert_type(h, jnp.int32)
    r = jnp.right_shift(u + 0x7FFF + (jnp.right_shift(u, 16) & 1), 16)
    return (r[:, :D // 2] & 0xFFFF) | (r[:, D // 2:] << 16)


def _agg_perm():
    perm = []
    for c in range(2):
        for q in range(2):
            base = 32 * c + 16 * q
            perm += [base + i for i in range(16)]
            perm += [64 + base + i for i in range(16)]
    return perm


_AGG_PERM = _agg_perm()


def _ln(x, g, b):
    m = jnp.mean(x, axis=-1, keepdims=True)
    v = jnp.mean((x - m) ** 2, axis=-1, keepdims=True)
    return (x - m) * lax.rsqrt(v + _EPS) * g + b


def _elu(x):
    return jnp.where(x > 0, x, jnp.exp(jnp.minimum(x, 0.0)) - 1.0)


def _first_body(x_ref, g1_ref, b1_ref, w1_ref, bw1_ref, gn_ref, bn_ref,
                o_ref, op_ref):
    x = x_ref[...]
    h = _ln(x, g1_ref[...], b1_ref[...])
    h = _elu(lax.dot_general(h, w1_ref[...], (((1,), (0,)), ((), ())),
                             precision=_PREC) + bw1_ref[...])
    h = _ln(h, gn_ref[...], bn_ref[...])
    o_ref[...] = h
    op_ref[...] = _pack_h(h)


def _layer_body(p0_ref, p1_ref, h_ref, wc_ref, bc_ref, gn_ref, bn_ref,
                o_ref, op_ref):
    agg = jnp.concatenate([p0_ref[...], p1_ref[...]], axis=1)
    conv = lax.dot_general(agg, wc_ref[...], (((1,), (0,)), ((), ())),
                           precision=_PREC) + bc_ref[...]
    h = _ln(conv, gn_ref[...], bn_ref[...]) + h_ref[...]
    o_ref[...] = h
    op_ref[...] = _pack_h(h)


def _last_body(p0_ref, p1_ref, h_ref, wc_ref, bc_ref, gn_ref, bn_ref,
               gf_ref, bf_ref, w1_ref, b1_ref, w2_ref, b2_ref, o_ref):
    agg = jnp.concatenate([p0_ref[...], p1_ref[...]], axis=1)
    conv = lax.dot_general(agg, wc_ref[...], (((1,), (0,)), ((), ())),
                           precision=_PREC) + bc_ref[...]
    h = _ln(conv, gn_ref[...], bn_ref[...]) + h_ref[...]
    t = _ln(h, gf_ref[...], bf_ref[...])
    t = _elu(lax.dot_general(t, w1_ref[...], (((1,), (0,)), ((), ())),
                             precision=_PREC) + b1_ref[...])
    o_ref[...] = lax.dot_general(t, w2_ref[...], (((1,), (0,)), ((), ())),
                                 precision=_PREC) + b2_ref[...]


def _final_body(h_ref, gf_ref, bf_ref, w1_ref, b1_ref, w2_ref, b2_ref, o_ref):
    t = _ln(h_ref[...], gf_ref[...], bf_ref[...])
    t = _elu(lax.dot_general(t, w1_ref[...], (((1,), (0,)), ((), ())),
                             precision=_PREC) + b1_ref[...])
    o_ref[...] = lax.dot_general(t, w2_ref[...], (((1,), (0,)), ((), ())),
                                 precision=_PREC) + b2_ref[...]


_BLK = 1000
_GRID = N // _BLK


def _row_spec():
    return pl.BlockSpec((_BLK, D), lambda i: (i, 0))


def _vec_spec():
    return pl.BlockSpec((1, D), lambda i: (0, 0))


def _mat_spec():
    return pl.BlockSpec((D, D), lambda i: (0, 0))


def _tc_first(x, g1, b1, W1, bw1, gn, bn):
    return pl.pallas_call(
        _first_body,
        grid=(_GRID,),
        in_specs=[_row_spec(), _vec_spec(), _vec_spec(), _mat_spec(),
                  _vec_spec(), _vec_spec(), _vec_spec()],
        out_specs=[_row_spec(), _half_spec()],
        out_shape=[jax.ShapeDtypeStruct((N, D), jnp.float32),
                   jax.ShapeDtypeStruct((N, D // 2), jnp.int32)],
    )(x, g1, b1, W1, bw1, gn, bn)


def _half_spec():
    return pl.BlockSpec((_BLK, DH), lambda i: (i, 0))


def _tc_layer(p0, p1, h, Wc, bc, gn, bn):
    return pl.pallas_call(
        _layer_body,
        grid=(_GRID,),
        in_specs=[_half_spec(), _half_spec(), _row_spec(), _mat_spec(),
                  _vec_spec(), _vec_spec(), _vec_spec()],
        out_specs=[_row_spec(), _half_spec()],
        out_shape=[jax.ShapeDtypeStruct((N, D), jnp.float32),
                   jax.ShapeDtypeStruct((N, D // 2), jnp.int32)],
    )(p0, p1, h, Wc, bc, gn, bn)


def _tc_final(h, gf, bf, Wf1, bwf1, Wf2, bwf2):
    return pl.pallas_call(
        _final_body,
        grid=(_GRID,),
        in_specs=[_row_spec(), _vec_spec(), _vec_spec(), _mat_spec(),
                  _vec_spec(), _mat_spec(), _vec_spec()],
        out_specs=_row_spec(),
        out_shape=jax.ShapeDtypeStruct((N, D), jnp.float32),
    )(h, gf, bf, Wf1, bwf1, Wf2, bwf2)


def _tc_last(p0, p1, h, Wc, bc, gn, bn, gf, bf, Wf1, bwf1, Wf2, bwf2):
    return pl.pallas_call(
        _last_body,
        grid=(_GRID,),
        in_specs=[_half_spec(), _half_spec(), _row_spec(), _mat_spec(),
                  _vec_spec(), _vec_spec(), _vec_spec(), _vec_spec(),
                  _vec_spec(), _mat_spec(), _vec_spec(), _mat_spec(),
                  _vec_spec()],
        out_specs=_row_spec(),
        out_shape=jax.ShapeDtypeStruct((N, D), jnp.float32),
    )(p0, p1, h, Wc, bc, gn, bn, gf, bf, Wf1, bwf1, Wf2, bwf2)


def kernel(x, edge_index, edge_weight, g1, b1, W1, bw1, gn, bn,
           Wc0, bc0, Wc1, bc1, Wc2, bc2, gf, bf, Wf1, bwf1, Wf2, bwf2):
    src = edge_index[0]
    dst = edge_index[1]
    zeros = jnp.zeros((N, DH), jnp.float32)

    r = lambda v: v.reshape(1, D)
    h, hp = _tc_first(x, r(g1), r(b1), W1, r(bw1), r(gn), r(bn))

    perm = jnp.asarray(_AGG_PERM, jnp.int32)
    for Wc, bc in ((Wc0, bc0), (Wc1, bc1)):
        parts = _sc_agg(hp, src, dst, edge_weight, zeros)
        h, hp = _tc_layer(parts[0], parts[1], h, Wc[perm], r(bc),
                          r(gn), r(bn))

    parts = _sc_agg(hp, src, dst, edge_weight, zeros)
    return _tc_last(parts[0], parts[1], h, Wc2[perm], r(bc2), r(gn), r(bn),
                    r(gf), r(bf), Wf1, r(bwf1), Wf2, r(bwf2))
